# baseline (speedup 1.0000x reference)
import functools

import jax
import jax.numpy as jnp
from jax import lax
from jax.experimental import pallas as pl
from jax.experimental.pallas import tpu as pltpu
from jax.experimental.pallas import tpu_sc as plsc

N = 32768
NTN = 4096
NT = 64
DIM = 64
SUB = 50000
TOKV = 100000

NC = 2
NS = 16
NW = NC * NS
RW = N // NW
CH = 128
NCH = RW // CH

_mesh = plsc.VectorSubcoreMesh(core_axis_name="c", subcore_axis_name="s")


@functools.partial(
    pl.kernel,
    mesh=_mesh,
    out_type=jax.ShapeDtypeStruct((N, DIM), jnp.float32),
    scratch_types=[
        pltpu.VMEM((NCH, CH), jnp.int32),
        pltpu.VMEM((RW // 2, DIM), jnp.float32),
        pltpu.VMEM((RW // 2, DIM), jnp.float32),
        pltpu.SemaphoreType.DMA,
        pltpu.SemaphoreType.DMA,
    ],
    compiler_params=pltpu.CompilerParams(use_tc_tiling_on_sc=False),
)
def _gather_sc(tab, idx_hbm, rows_out, idx_v, buf_a, buf_b, gsem, wsem):
    wid = lax.axis_index("s") * NC + lax.axis_index("c")
    base = wid * RW
    half = RW // 2
    for c in range(NCH):
        pltpu.sync_copy(idx_hbm.at[pl.ds(base + c * CH, CH)], idx_v.at[c])
    bufs = (buf_a, buf_b)
    wbs = []
    for p in range(2):
        buf = bufs[p % 2]
        cps = [
            pltpu.async_copy(tab.at[idx_v.at[p * 4 + c]],
                             buf.at[pl.ds(c * CH, CH)], gsem)
            for c in range(4)
        ]
        for cp in cps:
            cp.wait()
        wbs.append(pltpu.async_copy(
            buf, rows_out.at[pl.ds(base + p * half, half)], wsem))
    for wb in wbs:
        wb.wait()


_BT = 8192
_TBLK = -(-TOKV // _BT)


def _tpose_body(x_ref, o_ref):
    xt = jnp.transpose(x_ref[...])
    o_ref[...] = jnp.concatenate([xt[:_BT // 2], xt[_BT // 2:]], axis=1)


def _tpose_tc(tab_t):
    return pl.pallas_call(
        _tpose_body,
        grid=(_TBLK,),
        in_specs=[pl.BlockSpec((DIM, _BT), lambda j: (0, j))],
        out_specs=pl.BlockSpec((_BT // 2, 2 * DIM), lambda j: (j, 0)),
        out_shape=jax.ShapeDtypeStruct((TOKV // 2, 2 * DIM), jnp.float32),
        compiler_params=pltpu.CompilerParams(
            dimension_semantics=("arbitrary",)),
    )(tab_t)


_BN = 4096


def _conv_body(pt_ref, pk_ref, eta_ref, a_ref, b_ref, bh_ref,
               wt_ref, wl_ref, wr_ref, out_ref):
    i = pl.program_id(0)
    n_blk = N // _BN
    he = (jnp.dot(pt_ref[:, :DIM], a_ref[...], preferred_element_type=jnp.float32)
          + jnp.dot(pk_ref[:, :DIM], b_ref[...], preferred_element_type=jnp.float32)
          + bh_ref[...])
    ho = (jnp.dot(pt_ref[:, DIM:], a_ref[...], preferred_element_type=jnp.float32)
          + jnp.dot(pk_ref[:, DIM:], b_ref[...], preferred_element_type=jnp.float32)
          + bh_ref[...])
    hbe = he.astype(jnp.bfloat16)
    hbo = ho.astype(jnp.bfloat16)
    rows = lax.broadcasted_iota(jnp.int32, (6 * n_blk, 1), 0)

    def col(x, par):
        m = (rows == (2 * x + par) * n_blk + i).astype(jnp.float32)
        return lax.dot_general(eta_ref[...], m, (((0,), (0,)), ((), ())),
                               preferred_element_type=jnp.float32)

    conv_e = (
        col(0, 0) * jnp.dot(hbe, wt_ref[...], preferred_element_type=jnp.float32)
        + col(1, 0) * jnp.dot(hbe, wl_ref[...], preferred_element_type=jnp.float32)
        + col(2, 0) * jnp.dot(hbe, wr_ref[...], preferred_element_type=jnp.float32))
    conv_o = (
        col(0, 1) * jnp.dot(hbo, wt_ref[...], preferred_element_type=jnp.float32)
        + col(1, 1) * jnp.dot(hbo, wl_ref[...], preferred_element_type=jnp.float32)
        + col(2, 1) * jnp.dot(hbo, wr_ref[...], preferred_element_type=jnp.float32))
    out_ref[...] = jnp.concatenate([conv_e, conv_o], axis=1)


def _conv_tc(pt, pk, eta_all, a, b, bh, wt, wl, wr):
    n_blk = N // _BN
    row = lambda i: (i, 0)
    full = lambda i: (0, 0)
    return pl.pallas_call(
        _conv_body,
        grid=(n_blk,),
        in_specs=[
            pl.BlockSpec((_BN // 2, 2 * DIM), row),
            pl.BlockSpec((_BN // 2, 2 * DIM), row),
            pl.BlockSpec((6 * n_blk, _BN // 2), lambda i: (0, 0)),
            pl.BlockSpec((DIM, DIM), full),
            pl.BlockSpec((DIM, DIM), full),
            pl.BlockSpec((1, DIM), full),
            pl.BlockSpec((DIM, DIM), full),
            pl.BlockSpec((DIM, DIM), full),
            pl.BlockSpec((DIM, DIM), full),
        ],
        out_specs=pl.BlockSpec((_BN // 2, 2 * DIM), row),
        out_shape=jax.ShapeDtypeStruct((N // 2, 2 * DIM), jnp.float32),
        compiler_params=pltpu.CompilerParams(
            dimension_semantics=("arbitrary",)),
    )(pt, pk, eta_all, a, b, bh, wt, wl, wr)


@functools.partial(
    pl.kernel,
    mesh=_mesh,
    out_type=jax.ShapeDtypeStruct((NC, NTN, DIM), jnp.float32),
    scratch_types=[
        pltpu.VMEM((NCH, CH), jnp.int32),
        pltpu.VMEM((RW, DIM), jnp.float32),
        pltpu.VMEM_SHARED((NTN, DIM), jnp.float32),
        pltpu.SemaphoreType.DMA,
    ],
    compiler_params=pltpu.CompilerParams(use_tc_tiling_on_sc=False),
)
def _scatter_sc(conv_hbm, nidx_hbm, zeros_hbm, out_hbm,
                idx_v, rows_v, acc_sh, sem):
    cid = lax.axis_index("c")
    sid = lax.axis_index("s")
    wid = sid * NC + cid
    seg = NTN // NS
    pltpu.sync_copy(zeros_hbm, acc_sh.at[pl.ds(sid * seg, seg)])
    plsc.subcore_barrier()
    for c in range(NCH):
        pltpu.sync_copy(nidx_hbm.at[pl.ds(wid * RW + c * CH, CH)],
                        idx_v.at[c])
    pltpu.sync_copy(conv_hbm.at[pl.ds(wid * RW, RW)], rows_v)
    for c in range(NCH):
        pltpu.sync_copy(rows_v.at[pl.ds(c * CH, CH)],
                        acc_sh.at[idx_v.at[c]], add=True)
    plsc.subcore_barrier()
    pltpu.sync_copy(acc_sh.at[pl.ds(sid * seg, seg)],
                    out_hbm.at[cid].at[pl.ds(sid * seg, seg)])


_BK = 4096
_KBLK = -(-SUB // _BK)


def _tail_body(pre_ref, tree_ref, bc_ref, alpha_ref, wo_ref, bo_ref,
               out_ref, cv_ref):
    @pl.when(pl.program_id(0) == 0)
    def _():
        pp = pre_ref[0] + pre_ref[1]
        pre = jnp.concatenate([pp[:, :DIM], pp[:, DIM:]], axis=0)
        node_emb = jnp.tanh(pre + bc_ref[0, 0])
        onehot = (tree_ref[...] ==
                  lax.broadcasted_iota(jnp.int32, (NT, NTN), 0)
                  ).astype(jnp.float32)
        interT = lax.dot_general(alpha_ref[...], node_emb,
                                 (((1,), (1,)), ((), ())),
                                 preferred_element_type=jnp.float32)
        seg_max = jnp.max(jnp.where(onehot > 0.5, interT, -1e30),
                          axis=1, keepdims=True)
        maxn = lax.dot_general(seg_max, onehot, (((0,), (0,)), ((), ())),
                               preferred_element_type=jnp.float32)
        ex = jnp.exp(interT - maxn)
        denom = lax.dot_general(onehot, ex, (((1,), (1,)), ((), ())),
                                preferred_element_type=jnp.float32)
        denn = lax.dot_general(denom, onehot, (((0,), (0,)), ((), ())),
                               preferred_element_type=jnp.float32)
        wts = onehot * (ex / denn)
        cv_ref[...] = lax.dot_general(wts, node_emb,
                                      (((1,), (0,)), ((), ())),
                                      preferred_element_type=jnp.float32)

    out_ref[...] = (lax.dot_general(cv_ref[...], wo_ref[...],
                                    (((1,), (0,)), ((), ())),
                                    preferred_element_type=jnp.float32)
                    + bo_ref[...])


def _tail_tc(pre2, tree, bc, alpha_r, wo, bo):
    return pl.pallas_call(
        _tail_body,
        grid=(_KBLK,),
        in_specs=[
            pl.BlockSpec((NC, NTN // 2, 2 * DIM), lambda j: (0, 0, 0)),
            pl.BlockSpec((1, NTN), lambda j: (0, 0)),
            pl.BlockSpec((1, 1), lambda j: (0, 0)),
            pl.BlockSpec((1, DIM), lambda j: (0, 0)),
            pl.BlockSpec((DIM, _BK), lambda j: (0, j)),
            pl.BlockSpec((1, _BK), lambda j: (0, j)),
        ],
        out_specs=pl.BlockSpec((NT, _BK), lambda j: (0, j)),
        out_shape=jax.ShapeDtypeStruct((NT, SUB), jnp.float32),
        scratch_shapes=[pltpu.VMEM((NT, DIM), jnp.float32)],
        compiler_params=pltpu.CompilerParams(
            dimension_semantics=("arbitrary",)),
    )(pre2, tree, bc, alpha_r, wo, bo)


def kernel(type_batch, token_batch, node_indices, eta_t, eta_l, eta_r,
           tree_indices, emb_type, emb_token, W_h, b_h, w_t, w_l, w_r,
           bias_conv, alpha, W_out, b_out):
    f32 = jnp.float32
    tb = type_batch.astype(jnp.int32)
    kb0 = token_batch.astype(jnp.int32)
    blk = kb0 // _BT
    r = kb0 % _BT
    kb = blk * _BT + jnp.where(r >= _BT // 2, 2 * (r - _BT // 2) + 1, 2 * r)
    ni = node_indices.astype(jnp.int32)
    ti0 = tree_indices.astype(jnp.int32)
    ti = jnp.concatenate([ti0[0::2], ti0[1::2]]).reshape(1, NTN)
    eta_all = jnp.concatenate(
        [eta_t.astype(f32)[0::2], eta_t.astype(f32)[1::2],
         eta_l.astype(f32)[0::2], eta_l.astype(f32)[1::2],
         eta_r.astype(f32)[0::2], eta_r.astype(f32)[1::2]]
    ).reshape(6 * (N // _BN), _BN // 2)
    a = W_h[:, :DIM].T.astype(f32)
    b = W_h[:, DIM:].T.astype(f32)
    bh = b_h.astype(f32).reshape(1, DIM)
    wt = w_t.T.astype(jnp.bfloat16)
    wl = w_l.T.astype(jnp.bfloat16)
    wr = w_r.T.astype(jnp.bfloat16)
    zeros = jnp.zeros((NTN // NS, DIM), f32)

    tok_flat = _tpose_tc(emb_token.T.astype(f32)).reshape(TOKV, DIM)
    pt = _gather_sc(emb_type.astype(f32), tb).reshape(N // 2, 2 * DIM)
    pk = _gather_sc(tok_flat, kb).reshape(N // 2, 2 * DIM)
    conv_packed = _conv_tc(pt, pk, eta_all, a, b, bh, wt, wl, wr)
    conv = conv_packed.reshape(N, DIM)
    pre2 = _scatter_sc(conv, ni, zeros).reshape(NC, NTN // 2, 2 * DIM)
    logits = _tail_tc(pre2, ti, bias_conv.reshape(1, 1).astype(f32),
                      alpha.reshape(1, DIM).astype(f32),
                      W_out.T.astype(f32),
                      b_out.reshape(1, SUB).astype(f32))
    return logits


# speedup vs baseline: 1.0356x; 1.0356x over previous
import functools

import jax
import jax.numpy as jnp
from jax import lax
from jax.experimental import pallas as pl
from jax.experimental.pallas import tpu as pltpu
from jax.experimental.pallas import tpu_sc as plsc

N = 32768
NTN = 4096
NT = 64
DIM = 64
SUB = 50000
TOKV = 100000

NC = 2
NS = 16
NW = NC * NS
RW = N // NW
CH = 128
NCH = RW // CH

_mesh = plsc.VectorSubcoreMesh(core_axis_name="c", subcore_axis_name="s")


_BT = 8192
_TBLK = -(-TOKV // _BT)


def _tpose_body(x_ref, o_ref):
    xt = jnp.transpose(x_ref[...])
    o_ref[...] = jnp.concatenate([xt[:_BT // 2], xt[_BT // 2:]], axis=1)


def _tpose_tc(tab_t):
    return pl.pallas_call(
        _tpose_body,
        grid=(_TBLK,),
        in_specs=[pl.BlockSpec((DIM, _BT), lambda j: (0, j))],
        out_specs=pl.BlockSpec((_BT // 2, 2 * DIM), lambda j: (j, 0)),
        out_shape=jax.ShapeDtypeStruct((TOKV // 2, 2 * DIM), jnp.float32),
        compiler_params=pltpu.CompilerParams(
            dimension_semantics=("arbitrary",)),
    )(tab_t)


@functools.partial(
    pl.kernel,
    mesh=_mesh,
    out_type=jax.ShapeDtypeStruct((N, 2 * DIM), jnp.float32),
    scratch_types=[
        pltpu.VMEM((2 * NCH, CH), jnp.int32),
        pltpu.VMEM((RW // 2, DIM), jnp.float32),
        pltpu.VMEM((RW // 2, DIM), jnp.float32),
        pltpu.SemaphoreType.DMA,
        pltpu.SemaphoreType.DMA,
    ],
    compiler_params=pltpu.CompilerParams(use_tc_tiling_on_sc=False),
)
def _gather_sc(typ_tab, tok_tab, typ_idx, tok_idx, p_out,
               idx_v, buf_a, buf_b, gsem, wsem):
    wid = lax.axis_index("s") * NC + lax.axis_index("c")
    base = wid * RW
    half = RW // 2
    for c in range(NCH):
        pltpu.sync_copy(typ_idx.at[pl.ds(base + c * CH, CH)], idx_v.at[c])
        pltpu.sync_copy(tok_idx.at[pl.ds(base + c * CH, CH)],
                        idx_v.at[NCH + c])
    bufs = (buf_a, buf_b)
    phases = [(typ_tab, 0, 0, 0), (typ_tab, 4, 0, half),
              (tok_tab, 8, DIM, 0), (tok_tab, 12, DIM, half)]
    wbs = []
    for p, (tab, ir, col, roff) in enumerate(phases):
        buf = bufs[p % 2]
        if len(wbs) >= 2:
            wbs[p - 2].wait()
        cps = [
            pltpu.async_copy(tab.at[idx_v.at[ir + c]],
                             buf.at[pl.ds(c * CH, CH)], gsem)
            for c in range(4)
        ]
        for cp in cps:
            cp.wait()
        wbs.append(pltpu.async_copy(
            buf, p_out.at[pl.ds(base + roff, half), pl.ds(col, DIM)], wsem))
    wbs[2].wait()
    wbs[3].wait()


_BN = 4096


def _conv_body(p_ref, eta_ref, wh_ref, bh_ref, wt_ref, wl_ref, wr_ref,
               out_ref):
    i = pl.program_id(0)
    n_blk = N // _BN
    hidden = (jnp.dot(p_ref[...], wh_ref[...],
                      preferred_element_type=jnp.float32)
              + bh_ref[...])
    hb = hidden.astype(jnp.bfloat16)
    eta = eta_ref[...]
    lane = lax.broadcasted_iota(jnp.int32, (1, 3 * n_blk), 1)

    def col(x):
        m = (lane == x * n_blk + i).astype(jnp.float32)
        return jnp.sum(eta * m, axis=1, keepdims=True)

    conv = (
        col(0) * jnp.dot(hb, wt_ref[...], preferred_element_type=jnp.float32)
        + col(1) * jnp.dot(hb, wl_ref[...], preferred_element_type=jnp.float32)
        + col(2) * jnp.dot(hb, wr_ref[...], preferred_element_type=jnp.float32))
    out_ref[...] = jnp.concatenate(
        [conv[:_BN // 2], conv[_BN // 2:]], axis=1)


def _conv_tc(p, eta_all, wh, bh, wt, wl, wr):
    n_blk = N // _BN
    row = lambda i: (i, 0)
    full = lambda i: (0, 0)
    return pl.pallas_call(
        _conv_body,
        grid=(n_blk,),
        in_specs=[
            pl.BlockSpec((_BN, 2 * DIM), row),
            pl.BlockSpec((_BN, 3 * n_blk), lambda i: (0, 0)),
            pl.BlockSpec((2 * DIM, DIM), full),
            pl.BlockSpec((1, DIM), full),
            pl.BlockSpec((DIM, DIM), full),
            pl.BlockSpec((DIM, DIM), full),
            pl.BlockSpec((DIM, DIM), full),
        ],
        out_specs=pl.BlockSpec((_BN // 2, 2 * DIM), row),
        out_shape=jax.ShapeDtypeStruct((N // 2, 2 * DIM), jnp.float32),
        compiler_params=pltpu.CompilerParams(
            dimension_semantics=("arbitrary",)),
    )(p, eta_all, wh, bh, wt, wl, wr)


@functools.partial(
    pl.kernel,
    mesh=_mesh,
    out_type=jax.ShapeDtypeStruct((NC, NTN, DIM), jnp.float32),
    scratch_types=[
        pltpu.VMEM((NCH, CH), jnp.int32),
        pltpu.VMEM((RW, DIM), jnp.float32),
        pltpu.VMEM_SHARED((NTN, DIM), jnp.float32),
        pltpu.SemaphoreType.DMA,
    ],
    compiler_params=pltpu.CompilerParams(use_tc_tiling_on_sc=False),
)
def _scatter_sc(conv_hbm, nidx_hbm, zeros_hbm, out_hbm,
                idx_v, rows_v, acc_sh, sem):
    cid = lax.axis_index("c")
    sid = lax.axis_index("s")
    wid = sid * NC + cid
    seg = NTN // NS
    pltpu.sync_copy(zeros_hbm, acc_sh.at[pl.ds(sid * seg, seg)])
    plsc.subcore_barrier()
    for c in range(NCH):
        pltpu.sync_copy(nidx_hbm.at[pl.ds(wid * RW + c * CH, CH)],
                        idx_v.at[c])
    pltpu.sync_copy(conv_hbm.at[pl.ds(wid * RW, RW)], rows_v)
    for c in range(NCH):
        pltpu.sync_copy(rows_v.at[pl.ds(c * CH, CH)],
                        acc_sh.at[idx_v.at[c]], add=True)
    plsc.subcore_barrier()
    pltpu.sync_copy(acc_sh.at[pl.ds(sid * seg, seg)],
                    out_hbm.at[cid].at[pl.ds(sid * seg, seg)])


_BK = 4096
_KBLK = -(-SUB // _BK)


def _tail_body(pre_ref, tree_ref, bc_ref, alpha_ref, wo_ref, bo_ref,
               out_ref, cv_ref):
    @pl.when(pl.program_id(0) == 0)
    def _():
        pp = pre_ref[0] + pre_ref[1]
        pre = jnp.concatenate([pp[:, :DIM], pp[:, DIM:]], axis=0)
        node_emb = jnp.tanh(pre + bc_ref[0, 0])
        onehot = (tree_ref[...] ==
                  lax.broadcasted_iota(jnp.int32, (NT, NTN), 0)
                  ).astype(jnp.float32)
        interT = lax.dot_general(alpha_ref[...], node_emb,
                                 (((1,), (1,)), ((), ())),
                                 preferred_element_type=jnp.float32)
        seg_max = jnp.max(jnp.where(onehot > 0.5, interT, -1e30),
                          axis=1, keepdims=True)
        maxn = lax.dot_general(seg_max, onehot, (((0,), (0,)), ((), ())),
                               preferred_element_type=jnp.float32)
        ex = jnp.exp(interT - maxn)
        denom = lax.dot_general(onehot, ex, (((1,), (1,)), ((), ())),
                                preferred_element_type=jnp.float32)
        denn = lax.dot_general(denom, onehot, (((0,), (0,)), ((), ())),
                               preferred_element_type=jnp.float32)
        wts = onehot * (ex / denn)
        cv_ref[...] = lax.dot_general(wts, node_emb,
                                      (((1,), (0,)), ((), ())),
                                      preferred_element_type=jnp.float32)

    out_ref[...] = (lax.dot_general(cv_ref[...], wo_ref[...],
                                    (((1,), (0,)), ((), ())),
                                    preferred_element_type=jnp.float32)
                    + bo_ref[...])


def _tail_tc(pre2, tree, bc, alpha_r, wo, bo):
    return pl.pallas_call(
        _tail_body,
        grid=(_KBLK,),
        in_specs=[
            pl.BlockSpec((NC, NTN // 2, 2 * DIM), lambda j: (0, 0, 0)),
            pl.BlockSpec((1, NTN), lambda j: (0, 0)),
            pl.BlockSpec((1, 1), lambda j: (0, 0)),
            pl.BlockSpec((1, DIM), lambda j: (0, 0)),
            pl.BlockSpec((DIM, _BK), lambda j: (0, j)),
            pl.BlockSpec((1, _BK), lambda j: (0, j)),
        ],
        out_specs=pl.BlockSpec((NT, _BK), lambda j: (0, j)),
        out_shape=jax.ShapeDtypeStruct((NT, SUB), jnp.float32),
        scratch_shapes=[pltpu.VMEM((NT, DIM), jnp.float32)],
        compiler_params=pltpu.CompilerParams(
            dimension_semantics=("arbitrary",)),
    )(pre2, tree, bc, alpha_r, wo, bo)


def kernel(type_batch, token_batch, node_indices, eta_t, eta_l, eta_r,
           tree_indices, emb_type, emb_token, W_h, b_h, w_t, w_l, w_r,
           bias_conv, alpha, W_out, b_out):
    f32 = jnp.float32
    tb = type_batch.astype(jnp.int32)
    kb0 = token_batch.astype(jnp.int32)
    blk = kb0 // _BT
    r = kb0 % _BT
    kb = blk * _BT + jnp.where(r >= _BT // 2, 2 * (r - _BT // 2) + 1, 2 * r)
    ni = (node_indices.astype(jnp.int32)
          .reshape(N // _BN, 2, _BN // 2)
          .transpose(0, 2, 1)
          .reshape(N))
    ti0 = tree_indices.astype(jnp.int32)
    ti = jnp.concatenate([ti0[0::2], ti0[1::2]]).reshape(1, NTN)
    eta_all = jnp.concatenate(
        [eta_t.astype(f32).reshape(N // _BN, _BN).T,
         eta_l.astype(f32).reshape(N // _BN, _BN).T,
         eta_r.astype(f32).reshape(N // _BN, _BN).T], axis=1)
    wh = W_h.T.astype(f32)
    bh = b_h.astype(f32).reshape(1, DIM)
    wt = w_t.T.astype(jnp.bfloat16)
    wl = w_l.T.astype(jnp.bfloat16)
    wr = w_r.T.astype(jnp.bfloat16)
    zeros = jnp.zeros((NTN // NS, DIM), f32)

    tok_flat = _tpose_tc(emb_token.T.astype(f32)).reshape(TOKV, DIM)
    p = _gather_sc(emb_type.astype(f32), tok_flat, tb, kb)
    conv_packed = _conv_tc(p, eta_all, wh, bh, wt, wl, wr)
    conv = conv_packed.reshape(N, DIM)
    pre2 = _scatter_sc(conv, ni, zeros).reshape(NC, NTN // 2, 2 * DIM)
    logits = _tail_tc(pre2, ti, bias_conv.reshape(1, 1).astype(f32),
                      alpha.reshape(1, DIM).astype(f32),
                      W_out.T.astype(f32),
                      b_out.reshape(1, SUB).astype(f32))
    return logits


# speedup vs baseline: 1.0391x; 1.0033x over previous
import functools

import jax
import jax.numpy as jnp
from jax import lax
from jax.experimental import pallas as pl
from jax.experimental.pallas import tpu as pltpu
from jax.experimental.pallas import tpu_sc as plsc

N = 32768
NTN = 4096
NT = 64
DIM = 64
SUB = 50000
TOKV = 100000

NC = 2
NS = 16
NW = NC * NS
RW = N // NW
CH = 128
NCH = RW // CH


_BT = 8192
_TBLK = -(-TOKV // _BT)


def _tpose_body(x_ref, o_ref):
    xt = jnp.transpose(x_ref[...])
    o_ref[...] = jnp.concatenate([xt[:_BT // 2], xt[_BT // 2:]], axis=1)


def _tpose_tc(tab_t):
    return pl.pallas_call(
        _tpose_body,
        grid=(_TBLK,),
        in_specs=[pl.BlockSpec((DIM, _BT), lambda j: (0, j))],
        out_specs=pl.BlockSpec((_BT // 2, 2 * DIM), lambda j: (j, 0)),
        out_shape=jax.ShapeDtypeStruct((TOKV // 2, 2 * DIM), jnp.float32),
        compiler_params=pltpu.CompilerParams(
            dimension_semantics=("arbitrary",)),
    )(tab_t)


def _gather_body(typ_tab, tok_tab, typ_idx, tok_idx, p_out,
                 idx_v, buf_a, buf_b, gsem, wsem):
    wid = lax.axis_index("s") * NC + lax.axis_index("c")
    base = wid * RW
    half = RW // 2
    for c in range(NCH):
        pltpu.sync_copy(typ_idx.at[pl.ds(base + c * CH, CH)], idx_v.at[c])
        pltpu.sync_copy(tok_idx.at[pl.ds(base + c * CH, CH)],
                        idx_v.at[NCH + c])
    bufs = (buf_a, buf_b)
    phases = [(typ_tab, 0, 0, 0), (typ_tab, 4, 0, half),
              (tok_tab, 8, DIM, 0), (tok_tab, 12, DIM, half)]
    wbs = []
    for p, (tab, ir, col, roff) in enumerate(phases):
        buf = bufs[p % 2]
        if len(wbs) >= 2:
            wbs[p - 2].wait()
        cps = [
            pltpu.async_copy(tab.at[idx_v.at[ir + c]],
                             buf.at[pl.ds(c * CH, CH)], gsem)
            for c in range(4)
        ]
        for cp in cps:
            cp.wait()
        wbs.append(pltpu.async_copy(
            buf, p_out.at[pl.ds(base + roff, half), pl.ds(col, DIM)], wsem))
    wbs[2].wait()
    wbs[3].wait()


_BN = 4096


def _conv_body(p_ref, eta_ref, wh_ref, bh_ref, wt_ref, wl_ref, wr_ref,
               out_ref):
    i = pl.program_id(0)
    n_blk = N // _BN
    hidden = (jnp.dot(p_ref[...], wh_ref[...],
                      preferred_element_type=jnp.float32)
              + bh_ref[...])
    hb = hidden.astype(jnp.bfloat16)
    eta = eta_ref[...]
    lane = lax.broadcasted_iota(jnp.int32, (1, 3 * n_blk), 1)

    def col(x):
        m = (lane == x * n_blk + i).astype(jnp.float32)
        return jnp.sum(eta * m, axis=1, keepdims=True)

    conv = (
        col(0) * jnp.dot(hb, wt_ref[...], preferred_element_type=jnp.float32)
        + col(1) * jnp.dot(hb, wl_ref[...], preferred_element_type=jnp.float32)
        + col(2) * jnp.dot(hb, wr_ref[...], preferred_element_type=jnp.float32))
    out_ref[...] = jnp.concatenate(
        [conv[:_BN // 2], conv[_BN // 2:]], axis=1)


def _conv_tc(p, eta_all, wh, bh, wt, wl, wr):
    n_blk = N // _BN
    row = lambda i: (i, 0)
    full = lambda i: (0, 0)
    return pl.pallas_call(
        _conv_body,
        grid=(n_blk,),
        in_specs=[
            pl.BlockSpec((_BN, 2 * DIM), row),
            pl.BlockSpec((_BN, 3 * n_blk), lambda i: (0, 0)),
            pl.BlockSpec((2 * DIM, DIM), full),
            pl.BlockSpec((1, DIM), full),
            pl.BlockSpec((DIM, DIM), full),
            pl.BlockSpec((DIM, DIM), full),
            pl.BlockSpec((DIM, DIM), full),
        ],
        out_specs=pl.BlockSpec((_BN // 2, 2 * DIM), row),
        out_shape=jax.ShapeDtypeStruct((N // 2, 2 * DIM), jnp.float32),
        compiler_params=pltpu.CompilerParams(
            dimension_semantics=("arbitrary",)),
    )(p, eta_all, wh, bh, wt, wl, wr)


def _scatter_body(conv_hbm, nidx_hbm, zeros_hbm, out_hbm,
                  idx_v, rows_v, acc_sh, sem):
    cid = lax.axis_index("c")
    sid = lax.axis_index("s")
    wid = sid * NC + cid
    seg = NTN // NS
    pltpu.sync_copy(zeros_hbm, acc_sh.at[pl.ds(sid * seg, seg)])
    plsc.subcore_barrier()
    for c in range(NCH):
        pltpu.sync_copy(nidx_hbm.at[pl.ds(wid * RW + c * CH, CH)],
                        idx_v.at[c])
    pltpu.sync_copy(conv_hbm.at[pl.ds(wid * RW, RW)], rows_v)
    for c in range(NCH):
        pltpu.sync_copy(rows_v.at[pl.ds(c * CH, CH)],
                        acc_sh.at[idx_v.at[c]], add=True)
    plsc.subcore_barrier()
    pltpu.sync_copy(acc_sh.at[pl.ds(sid * seg, seg)],
                    out_hbm.at[cid].at[pl.ds(sid * seg, seg)])


_BK = 4096
_KBLK = -(-SUB // _BK)


def _tail_body(pre_ref, tree_ref, bc_ref, alpha_ref, wo_ref, bo_ref,
               out_ref, cv_ref):
    @pl.when(pl.program_id(0) == 0)
    def _():
        pp = pre_ref[0] + pre_ref[1]
        pre = jnp.concatenate([pp[:, :DIM], pp[:, DIM:]], axis=0)
        node_emb = jnp.tanh(pre + bc_ref[0, 0])
        onehot = (tree_ref[...] ==
                  lax.broadcasted_iota(jnp.int32, (NT, NTN), 0)
                  ).astype(jnp.float32)
        interT = lax.dot_general(alpha_ref[...], node_emb,
                                 (((1,), (1,)), ((), ())),
                                 preferred_element_type=jnp.float32)
        seg_max = jnp.max(jnp.where(onehot > 0.5, interT, -1e30),
                          axis=1, keepdims=True)
        maxn = lax.dot_general(seg_max, onehot, (((0,), (0,)), ((), ())),
                               preferred_element_type=jnp.float32)
        ex = jnp.exp(interT - maxn)
        denom = lax.dot_general(onehot, ex, (((1,), (1,)), ((), ())),
                                preferred_element_type=jnp.float32)
        denn = lax.dot_general(denom, onehot, (((0,), (0,)), ((), ())),
                               preferred_element_type=jnp.float32)
        wts = onehot * (ex / denn)
        cv_ref[...] = lax.dot_general(wts, node_emb,
                                      (((1,), (0,)), ((), ())),
                                      preferred_element_type=jnp.float32)

    out_ref[...] = (lax.dot_general(cv_ref[...], wo_ref[...],
                                    (((1,), (0,)), ((), ())),
                                    preferred_element_type=jnp.float32)
                    + bo_ref[...])


def _tail_tc(pre2, tree, bc, alpha_r, wo, bo):
    return pl.pallas_call(
        _tail_body,
        grid=(_KBLK,),
        in_specs=[
            pl.BlockSpec((NC, NTN // 2, 2 * DIM), lambda j: (0, 0, 0)),
            pl.BlockSpec((1, NTN), lambda j: (0, 0)),
            pl.BlockSpec((1, 1), lambda j: (0, 0)),
            pl.BlockSpec((1, DIM), lambda j: (0, 0)),
            pl.BlockSpec((DIM, _BK), lambda j: (0, j)),
            pl.BlockSpec((1, _BK), lambda j: (0, j)),
        ],
        out_specs=pl.BlockSpec((NT, _BK), lambda j: (0, j)),
        out_shape=jax.ShapeDtypeStruct((NT, SUB), jnp.float32),
        scratch_shapes=[pltpu.VMEM((NT, DIM), jnp.float32)],
        compiler_params=pltpu.CompilerParams(
            dimension_semantics=("arbitrary",)),
    )(pre2, tree, bc, alpha_r, wo, bo)


@functools.cache
def _sc_kernels():
    mesh = plsc.VectorSubcoreMesh(core_axis_name="c", subcore_axis_name="s")
    gather = functools.partial(
        pl.kernel,
        mesh=mesh,
        out_type=jax.ShapeDtypeStruct((N, 2 * DIM), jnp.float32),
        scratch_types=[
            pltpu.VMEM((2 * NCH, CH), jnp.int32),
            pltpu.VMEM((RW // 2, DIM), jnp.float32),
            pltpu.VMEM((RW // 2, DIM), jnp.float32),
            pltpu.SemaphoreType.DMA,
            pltpu.SemaphoreType.DMA,
        ],
        compiler_params=pltpu.CompilerParams(use_tc_tiling_on_sc=False),
    )(_gather_body)
    scatter = functools.partial(
        pl.kernel,
        mesh=mesh,
        out_type=jax.ShapeDtypeStruct((NC, NTN, DIM), jnp.float32),
        scratch_types=[
            pltpu.VMEM((NCH, CH), jnp.int32),
            pltpu.VMEM((RW, DIM), jnp.float32),
            pltpu.VMEM_SHARED((NTN, DIM), jnp.float32),
            pltpu.SemaphoreType.DMA,
        ],
        compiler_params=pltpu.CompilerParams(use_tc_tiling_on_sc=False),
    )(_scatter_body)
    return gather, scatter


def kernel(type_batch, token_batch, node_indices, eta_t, eta_l, eta_r,
           tree_indices, emb_type, emb_token, W_h, b_h, w_t, w_l, w_r,
           bias_conv, alpha, W_out, b_out):
    f32 = jnp.float32
    tb = type_batch.astype(jnp.int32)
    kb0 = token_batch.astype(jnp.int32)
    blk = kb0 // _BT
    r = kb0 % _BT
    kb = blk * _BT + jnp.where(r >= _BT // 2, 2 * (r - _BT // 2) + 1, 2 * r)
    ni = (node_indices.astype(jnp.int32)
          .reshape(N // _BN, 2, _BN // 2)
          .transpose(0, 2, 1)
          .reshape(N))
    ti0 = tree_indices.astype(jnp.int32)
    ti = jnp.concatenate([ti0[0::2], ti0[1::2]]).reshape(1, NTN)
    eta_all = jnp.concatenate(
        [eta_t.astype(f32).reshape(N // _BN, _BN).T,
         eta_l.astype(f32).reshape(N // _BN, _BN).T,
         eta_r.astype(f32).reshape(N // _BN, _BN).T], axis=1)
    wh = W_h.T.astype(f32)
    bh = b_h.astype(f32).reshape(1, DIM)
    wt = w_t.T.astype(jnp.bfloat16)
    wl = w_l.T.astype(jnp.bfloat16)
    wr = w_r.T.astype(jnp.bfloat16)
    zeros = jnp.zeros((NTN // NS, DIM), f32)

    gather_sc, scatter_sc = _sc_kernels()
    tok_flat = _tpose_tc(emb_token.T.astype(f32)).reshape(TOKV, DIM)
    p = gather_sc(emb_type.astype(f32), tok_flat, tb, kb)
    conv_packed = _conv_tc(p, eta_all, wh, bh, wt, wl, wr)
    conv = conv_packed.reshape(N, DIM)
    pre2 = scatter_sc(conv, ni, zeros).reshape(NC, NTN // 2, 2 * DIM)
    logits = _tail_tc(pre2, ti, bias_conv.reshape(1, 1).astype(f32),
                      alpha.reshape(1, DIM).astype(f32),
                      W_out.T.astype(f32),
                      b_out.reshape(1, SUB).astype(f32))
    return logits
